# baseline (speedup 1.0000x reference)
import functools

import jax
import jax.numpy as jnp
from jax import lax
from jax.experimental import pallas as pl
from jax.experimental.pallas import tpu as pltpu
from jax.experimental.pallas import tpu_sc as plsc

B, K, L, NUM_CLASSES, D = 16, 8, 2048, 1024, 128
NC, NS = 2, 16
KH = K // NC
THALF = KH * NUM_CLASSES
G = B * K
CH = 128
HC = CH // 2
GW = G // NC
NSTEP = 2 * GW
NB = 8
LOOK = 4
ROWS = B * K * L


def _emb_body(tables_hbm, idx_hbm, frame_hbm, out_hbm,
              tables_sp, idx_v, frame_v, rowbufs, gsems, ssems):
    c = lax.axis_index("c")
    p = lax.axis_index("s")

    tsl = THALF // NS
    pltpu.sync_copy(tables_hbm.at[pl.ds(c * THALF + p * tsl, tsl)],
                    tables_sp.at[pl.ds(p * tsl, tsl)])

    pltpu.sync_copy(idx_hbm.at[:, pl.ds(p * CH, CH)], idx_v)
    pltpu.sync_copy(frame_hbm.at[pl.ds(p * CH, CH)], frame_v)

    def grow(jj):
        return 8 * (jj // 4) + 4 * c + lax.rem(jj, 4)

    def adj(jj, carry):
        r = grow(jj)
        off = jnp.full((16,), lax.rem(jj, 4) * NUM_CLASSES, jnp.int32)
        for v in range(CH // 16):
            sl = (r, pl.ds(v * 16, 16))
            idx_v[sl] = idx_v[sl] + off
        return carry

    lax.fori_loop(0, GW, adj, 0)

    plsc.subcore_barrier()

    def gather_issue(j, b):
        jj, h = lax.rem(j, GW), j // GW
        idx_sl = idx_v.at[grow(jj), pl.ds(h * HC, HC)]
        pltpu.async_copy(tables_sp.at[idx_sl], rowbufs[b], gsems[b])

    def gather_wait(j, b):
        jj, h = lax.rem(j, GW), j // GW
        idx_sl = idx_v.at[grow(jj), pl.ds(h * HC, HC)]
        pltpu.make_async_copy(tables_sp.at[idx_sl], rowbufs[b],
                              gsems[b]).wait()

    def store_issue(j, b):
        jj, h = lax.rem(j, GW), j // GW
        base = grow(jj) * L + p * CH + h * HC
        pltpu.async_copy(rowbufs[b], out_hbm.at[pl.ds(base, HC)], ssems[b])

    def store_wait(b):
        pltpu.make_async_copy(rowbufs[b], out_hbm.at[pl.ds(0, HC)],
                              ssems[b]).wait()

    for b in range(LOOK):
        gather_issue(b, b)

    def step(i, carry):
        for q in range(2):
            j0 = NB * i + 4 * q
            b0 = 4 * q
            ob = 4 - b0
            gather_wait(j0, b0)
            gather_wait(j0 + 1, b0 + 1)
            gather_wait(j0 + 2, b0 + 2)
            gather_wait(j0 + 3, b0 + 3)

            if q == 0:
                @pl.when(i >= 1)
                def _():
                    for t in range(4):
                        store_wait(ob + t)
                for t in range(4):
                    gather_issue(j0 + 4 + t, ob + t)
            else:
                @pl.when(i < NSTEP // NB - 1)
                def _():
                    for t in range(4):
                        store_wait(ob + t)
                    for t in range(4):
                        gather_issue(j0 + 4 + t, ob + t)

            @plsc.parallel_loop(0, HC, step=1, unroll=2)
            def add_rows(r):
                fr = (j0 // GW) * HC + r
                for v in range(D // 16):
                    sl = (r, pl.ds(v * 16, 16))
                    fvec = frame_v[fr, pl.ds(v * 16, 16)]
                    for t in range(4):
                        plsc.addupdate(rowbufs[b0 + t].at[sl], fvec)

            for t in range(4):
                store_issue(j0 + t, b0 + t)
        return carry

    lax.fori_loop(0, NSTEP // NB, step, 0)
    for b in range(NB):
        store_wait(b)


@functools.partial(
    pl.kernel,
    mesh=plsc.VectorSubcoreMesh(core_axis_name="c", subcore_axis_name="s"),
    out_type=jax.ShapeDtypeStruct((ROWS, D), jnp.float32),
    scratch_types=(
        [pltpu.VMEM_SHARED((THALF, D), jnp.float32),
         pltpu.VMEM((G, CH), jnp.int32),
         pltpu.VMEM((CH, D), jnp.float32)]
        + [pltpu.VMEM((HC, D), jnp.float32)] * NB
        + [pltpu.SemaphoreType.DMA] * (2 * NB)
    ),
)
def _emb_kernel(tables_hbm, idx_hbm, frame_hbm, out_hbm,
                tables_sp, idx_v, frame_v, *bufs_and_sems):
    rowbufs = bufs_and_sems[:NB]
    gsems = bufs_and_sems[NB:2 * NB]
    ssems = bufs_and_sems[2 * NB:3 * NB]
    _emb_body(tables_hbm, idx_hbm, frame_hbm, out_hbm,
              tables_sp, idx_v, frame_v, rowbufs, gsems, ssems)


@jax.jit
def kernel(index, content_tables, frame_table):
    tables = content_tables.reshape(K * NUM_CLASSES, D)
    idx = index.reshape(G, L).astype(jnp.int32)
    out = _emb_kernel(tables, idx, frame_table[:L])
    return out.reshape(B, K, L, D)


# speedup vs baseline: 1.1208x; 1.1208x over previous
import functools

import jax
import jax.numpy as jnp
from jax import lax
from jax.experimental import pallas as pl
from jax.experimental.pallas import tpu as pltpu
from jax.experimental.pallas import tpu_sc as plsc

B, K, L, NUM_CLASSES, D = 16, 8, 2048, 1024, 128
NC, NS = 2, 16
KH = K // NC
THALF = KH * NUM_CLASSES
G = B * K
CH = 128
HC = CH // 2
GW = G // NC
NSTEP = 2 * GW
NB = 8
LOOK = 4
ROWS = B * K * L


def _emb_body(tables_hbm, idx_hbm, frame_hbm, out_hbm,
              tables_sp, idx_v, frame_v, rowbufs, gsems, ssems):
    c = lax.axis_index("c")
    p = lax.axis_index("s")

    tsl = THALF // NS
    tsrc = tables_hbm.at[pl.ds(c * THALF + p * tsl, tsl)]
    tdst = tables_sp.at[pl.ds(p * tsl, tsl)]
    isrc = idx_hbm.at[:, pl.ds(p * CH, CH)]
    fsrc = frame_hbm.at[pl.ds(p * CH, CH)]
    pltpu.async_copy(isrc, idx_v, ssems[0])
    pltpu.async_copy(tsrc, tables_sp.at[pl.ds(p * tsl, tsl)], ssems[1])
    pltpu.async_copy(fsrc, frame_v, ssems[2])
    pltpu.make_async_copy(isrc, idx_v, ssems[0]).wait()

    def grow(jj):
        return 8 * (jj // 4) + 4 * c + lax.rem(jj, 4)

    def adj(jj, carry):
        r = grow(jj)
        off = jnp.full((16,), lax.rem(jj, 4) * NUM_CLASSES, jnp.int32)
        for v in range(CH // 16):
            sl = (r, pl.ds(v * 16, 16))
            idx_v[sl] = idx_v[sl] + off
        return carry

    lax.fori_loop(0, GW, adj, 0)

    pltpu.make_async_copy(tsrc, tdst, ssems[1]).wait()
    pltpu.make_async_copy(fsrc, frame_v, ssems[2]).wait()
    plsc.subcore_barrier()

    def gather_issue(j, b):
        jj, h = lax.rem(j, GW), j // GW
        idx_sl = idx_v.at[grow(jj), pl.ds(h * HC, HC)]
        pltpu.async_copy(tables_sp.at[idx_sl], rowbufs[b], gsems[b])

    def gather_wait(j, b):
        jj, h = lax.rem(j, GW), j // GW
        idx_sl = idx_v.at[grow(jj), pl.ds(h * HC, HC)]
        pltpu.make_async_copy(tables_sp.at[idx_sl], rowbufs[b],
                              gsems[b]).wait()

    def store_issue(j, b):
        jj, h = lax.rem(j, GW), j // GW
        base = grow(jj) * L + p * CH + h * HC
        pltpu.async_copy(rowbufs[b], out_hbm.at[pl.ds(base, HC)], ssems[b])

    def store_wait(b):
        pltpu.make_async_copy(rowbufs[b], out_hbm.at[pl.ds(0, HC)],
                              ssems[b]).wait()

    for b in range(LOOK):
        gather_issue(b, b)

    def step(i, carry):
        for w in range(NB // 2):
            j0 = NB * i + 2 * w
            b0, b1 = 2 * w, 2 * w + 1
            nb0 = (2 * w + LOOK) % NB
            nb1 = nb0 + 1
            gather_wait(j0, b0)
            gather_wait(j0 + 1, b1)

            if w < LOOK // 2:
                @pl.when(i >= 1)
                def _():
                    store_wait(nb0)
                    store_wait(nb1)
                gather_issue(j0 + LOOK, nb0)
                gather_issue(j0 + LOOK + 1, nb1)
            else:
                @pl.when(i < NSTEP // NB - 1)
                def _():
                    store_wait(nb0)
                    store_wait(nb1)
                    gather_issue(j0 + LOOK, nb0)
                    gather_issue(j0 + LOOK + 1, nb1)

            @plsc.parallel_loop(0, HC, step=1, unroll=4)
            def add_rows(r):
                fr = (j0 // GW) * HC + r
                for v in range(D // 16):
                    sl = (r, pl.ds(v * 16, 16))
                    fvec = frame_v[fr, pl.ds(v * 16, 16)]
                    plsc.addupdate(rowbufs[b0].at[sl], fvec)
                    plsc.addupdate(rowbufs[b1].at[sl], fvec)

            store_issue(j0, b0)
            store_issue(j0 + 1, b1)
        return carry

    lax.fori_loop(0, NSTEP // NB, step, 0)
    for b in range(NB):
        store_wait(b)


@functools.partial(
    pl.kernel,
    mesh=plsc.VectorSubcoreMesh(core_axis_name="c", subcore_axis_name="s"),
    out_type=jax.ShapeDtypeStruct((ROWS, D), jnp.float32),
    scratch_types=(
        [pltpu.VMEM_SHARED((THALF, D), jnp.float32),
         pltpu.VMEM((G, CH), jnp.int32),
         pltpu.VMEM((CH, D), jnp.float32)]
        + [pltpu.VMEM((HC, D), jnp.float32)] * NB
        + [pltpu.SemaphoreType.DMA] * (2 * NB)
    ),
)
def _emb_kernel(tables_hbm, idx_hbm, frame_hbm, out_hbm,
                tables_sp, idx_v, frame_v, *bufs_and_sems):
    rowbufs = bufs_and_sems[:NB]
    gsems = bufs_and_sems[NB:2 * NB]
    ssems = bufs_and_sems[2 * NB:3 * NB]
    _emb_body(tables_hbm, idx_hbm, frame_hbm, out_hbm,
              tables_sp, idx_v, frame_v, rowbufs, gsems, ssems)


@jax.jit
def kernel(index, content_tables, frame_table):
    tables = content_tables.reshape(K * NUM_CLASSES, D)
    idx = index.reshape(G, L).astype(jnp.int32)
    out = _emb_kernel(tables, idx, frame_table[:L])
    return out.reshape(B, K, L, D)
